# baseline (speedup 1.0000x reference)
import functools

import jax
import jax.numpy as jnp
from jax import lax
from jax.experimental import pallas as pl
from jax.experimental.pallas import tpu as pltpu
from jax.experimental.pallas import tpu_sc as plsc

N = 10000
EMB = 128
DEG = 32
EG = N * DEG
NC = 2
NS = 16
NW = NC * NS
LANES = 16
K = 80
EPW = EG // NS
NCHUNK = EPW // K
NCHUNK_PAD = 256
CPB = 32
NBLK = NCHUNK_PAD // CPB
NZCH = N // K
MAXT = (NZCH + NS - 1) // NS


@functools.partial(
    pl.kernel,
    out_type=jax.ShapeDtypeStruct((NC, N, EMB), jnp.float32),
    mesh=plsc.VectorSubcoreMesh(core_axis_name="c", subcore_axis_name="s",
                                num_cores=NC, num_subcores=NS),
    scratch_types=[
        pltpu.VMEM_SHARED((N, EMB), jnp.float32),
        pltpu.VMEM((CPB, K), jnp.int32),
        pltpu.VMEM((CPB, K), jnp.int32),
        pltpu.VMEM((CPB, K), jnp.float32),
        pltpu.VMEM((2, K, EMB), jnp.float32),
        pltpu.SemaphoreType.DMA,
    ],
)
def _spmm_sc(x_hbm, src_hbm, dst_hbm, w_hbm, out_hbm, acc_sh,
             src_v, dst_v, w_v, rows_v, semg):
    c = lax.axis_index("c")
    s = lax.axis_index("s")
    wid = c * NS + s

    def zrow(r, _):
        for j in range(EMB // LANES):
            rows_v[0, r, pl.ds(j * LANES, LANES)] = jnp.zeros((LANES,),
                                                              jnp.float32)
        return 0
    lax.fori_loop(0, K, zrow, 0)
    for t in range(MAXT):
        idx = s + NS * t
        @pl.when(idx < NZCH)
        def _():
            off = pl.multiple_of(idx * K, 8)
            pltpu.sync_copy(rows_v.at[0], acc_sh.at[pl.ds(off, K)])
    plsc.subcore_barrier()

    def block(b, _):
        boff = pl.multiple_of(b * CPB, 8)
        pltpu.sync_copy(src_hbm.at[wid, pl.ds(boff, CPB)], src_v)
        pltpu.sync_copy(dst_hbm.at[wid, pl.ds(boff, CPB)], dst_v)
        pltpu.sync_copy(w_hbm.at[wid, pl.ds(boff, CPB)], w_v)

        pltpu.async_copy(x_hbm.at[src_v.at[0]], rows_v.at[0], semg)

        def chunk(ci, _):
            par = lax.rem(ci, 2)
            pltpu.make_async_copy(x_hbm.at[pl.ds(0, K)], rows_v.at[par],
                                  semg).wait()
            @pl.when(ci + 1 < CPB)
            def _():
                pltpu.async_copy(x_hbm.at[src_v.at[ci + 1]],
                                 rows_v.at[1 - par], semg)

            def group(g, _):
                wvec = w_v[ci, pl.ds(g * LANES, LANES)]
                dvec = dst_v[ci, pl.ds(g * LANES, LANES)]
                for el in range(LANES):
                    ws = wvec[el]
                    e = g * LANES + el
                    for j in range(EMB // LANES):
                        rows_v[par, e, pl.ds(j * LANES, LANES)] = (
                            rows_v[par, e, pl.ds(j * LANES, LANES)] * ws)
                roff = pl.multiple_of(g * LANES, LANES)
                pltpu.sync_copy(rows_v.at[par, pl.ds(roff, LANES)],
                                acc_sh.at[dvec], add=True)
                return 0
            lax.fori_loop(0, K // LANES, group, 0)
            return 0
        lax.fori_loop(0, CPB, chunk, 0)
        return 0
    lax.fori_loop(0, NBLK, block, 0)

    plsc.subcore_barrier()
    for t in range(MAXT):
        idx = s + NS * t
        @pl.when(idx < NZCH)
        def _():
            off = pl.multiple_of(idx * K, 8)
            pltpu.sync_copy(acc_sh.at[pl.ds(off, K)],
                            out_hbm.at[c, pl.ds(off, K)])


ROWS_BLK = 2000


def _dense_tc_body(p_ref, w_ref, b_ref, o_ref):
    z = jnp.dot(p_ref[0], w_ref[0], preferred_element_type=jnp.float32)
    z = z + b_ref[0, 0:1, :]
    z = jnp.maximum(z, 0.0)
    nrm = jnp.sqrt(jnp.sum(z * z, axis=1, keepdims=True))
    o_ref[0] = z / jnp.maximum(nrm, 1e-12)


def _dense_tc(p, wstack, bstack):
    bpad = jnp.broadcast_to(bstack[:, None, :], (NC, 8, EMB))
    return pl.pallas_call(
        _dense_tc_body,
        grid=(NC, N // ROWS_BLK),
        in_specs=[
            pl.BlockSpec((1, ROWS_BLK, EMB), lambda g, r: (g, r, 0)),
            pl.BlockSpec((1, EMB, EMB), lambda g, r: (g, 0, 0)),
            pl.BlockSpec((1, 8, EMB), lambda g, r: (g, 0, 0)),
        ],
        out_specs=pl.BlockSpec((1, ROWS_BLK, EMB), lambda g, r: (g, r, 0)),
        out_shape=jax.ShapeDtypeStruct((NC, N, EMB), jnp.float32),
    )(p, wstack, bpad)


def _prep_edges(a):
    a = a.reshape(NW, NCHUNK, K)
    return jnp.pad(a, ((0, 0), (0, NCHUNK_PAD - NCHUNK), (0, 0)))


def kernel(embedding_user, embedding_item, Wu0, bu0, Wu1, bu1, Wi0, bi0,
           Wi1, bi1, user_edge_weight, item_edge_weight, user_edge_index,
           item_edge_index):
    src = _prep_edges(jnp.concatenate([user_edge_index[0],
                                       item_edge_index[0] + N]))
    dst = _prep_edges(jnp.concatenate([user_edge_index[1],
                                       item_edge_index[1]]))
    w = _prep_edges(jnp.concatenate([user_edge_weight, item_edge_weight]))

    x = jnp.concatenate([embedding_user, embedding_item], axis=0)
    weights = [(jnp.stack([Wu0, Wi0]), jnp.stack([bu0, bi0])),
               (jnp.stack([Wu1, Wi1]), jnp.stack([bu1, bi1]))]
    for l in range(2):
        p = _spmm_sc(x, src, dst, w)
        y = _dense_tc(p, weights[l][0], weights[l][1])
        x = y.reshape(NC * N, EMB)
    return (y[0], y[1])


# speedup vs baseline: 1.5764x; 1.5764x over previous
import functools

import jax
import jax.numpy as jnp
from jax import lax
from jax.experimental import pallas as pl
from jax.experimental.pallas import tpu as pltpu
from jax.experimental.pallas import tpu_sc as plsc

N = 10000
EMB = 128
DEG = 32
EG = N * DEG
NC = 2
NS = 16
NW = NC * NS
LANES = 16
K = 80
EPW = EG // NS
NCHUNK = EPW // K
BLK_E = 2000
NBLK = EPW // BLK_E
CPB = BLK_E // K
NZCH = N // K
MAXT = (NZCH + NS - 1) // NS


@functools.partial(
    pl.kernel,
    out_type=jax.ShapeDtypeStruct((NC, N, EMB), jnp.float32),
    mesh=plsc.VectorSubcoreMesh(core_axis_name="c", subcore_axis_name="s",
                                num_cores=NC, num_subcores=NS),
    scratch_types=[
        pltpu.VMEM_SHARED((N, EMB), jnp.float32),
        pltpu.VMEM((BLK_E,), jnp.int32),
        pltpu.VMEM((BLK_E,), jnp.int32),
        pltpu.VMEM((BLK_E,), jnp.float32),
        pltpu.VMEM((2, K, EMB), jnp.float32),
        pltpu.SemaphoreType.DMA,
    ],
)
def _spmm_sc(x_hbm, src_hbm, dst_hbm, w_hbm, out_hbm, acc_sh,
             src_v, dst_v, w_v, rows_v, semg):
    c = lax.axis_index("c")
    s = lax.axis_index("s")
    ebase = (c * NS + s) * EPW

    def zrow(r, _):
        for j in range(EMB // LANES):
            rows_v[0, r, pl.ds(j * LANES, LANES)] = jnp.zeros((LANES,),
                                                              jnp.float32)
        return 0
    lax.fori_loop(0, K, zrow, 0)
    for t in range(MAXT):
        idx = s + NS * t
        @pl.when(idx < NZCH)
        def _():
            off = pl.multiple_of(idx * K, 8)
            pltpu.sync_copy(rows_v.at[0], acc_sh.at[pl.ds(off, K)])
    plsc.subcore_barrier()

    def block(b, _):
        boff = pl.multiple_of(b * BLK_E, 8)
        pltpu.sync_copy(src_hbm.at[pl.ds(ebase + boff, BLK_E)], src_v)
        pltpu.sync_copy(dst_hbm.at[pl.ds(ebase + boff, BLK_E)], dst_v)
        pltpu.sync_copy(w_hbm.at[pl.ds(ebase + boff, BLK_E)], w_v)

        pltpu.async_copy(x_hbm.at[src_v.at[pl.ds(0, K)]], rows_v.at[0], semg)

        def chunk(ci, _):
            par = lax.rem(ci, 2)
            pltpu.make_async_copy(x_hbm.at[pl.ds(0, K)], rows_v.at[par],
                                  semg).wait()
            @pl.when(ci + 1 < CPB)
            def _():
                koff2 = pl.multiple_of((ci + 1) * K, 8)
                pltpu.async_copy(x_hbm.at[src_v.at[pl.ds(koff2, K)]],
                                 rows_v.at[1 - par], semg)

            def group(g, _):
                goff = pl.multiple_of(ci * K + g * LANES, LANES)
                wvec = w_v[pl.ds(goff, LANES)]
                dvec = dst_v[pl.ds(goff, LANES)]
                for el in range(LANES):
                    ws = wvec[el]
                    e = g * LANES + el
                    for j in range(EMB // LANES):
                        rows_v[par, e, pl.ds(j * LANES, LANES)] = (
                            rows_v[par, e, pl.ds(j * LANES, LANES)] * ws)
                roff = pl.multiple_of(g * LANES, LANES)
                pltpu.sync_copy(rows_v.at[par, pl.ds(roff, LANES)],
                                acc_sh.at[dvec], add=True)
                return 0
            lax.fori_loop(0, K // LANES, group, 0)
            return 0
        lax.fori_loop(0, CPB, chunk, 0)
        return 0
    lax.fori_loop(0, NBLK, block, 0)

    plsc.subcore_barrier()
    for t in range(MAXT):
        idx = s + NS * t
        @pl.when(idx < NZCH)
        def _():
            off = pl.multiple_of(idx * K, 8)
            pltpu.sync_copy(acc_sh.at[pl.ds(off, K)],
                            out_hbm.at[c, pl.ds(off, K)])


ROWS_BLK = 2000


def _dense_tc_body(p_ref, w_ref, b_ref, o_ref):
    z = jnp.dot(p_ref[0], w_ref[0], preferred_element_type=jnp.float32)
    z = z + b_ref[0, 0:1, :]
    z = jnp.maximum(z, 0.0)
    nrm = jnp.sqrt(jnp.sum(z * z, axis=1, keepdims=True))
    o_ref[0] = z / jnp.maximum(nrm, 1e-12)


def _dense_tc(p, wstack, bstack):
    bpad = jnp.broadcast_to(bstack[:, None, :], (NC, 8, EMB))
    return pl.pallas_call(
        _dense_tc_body,
        grid=(NC, N // ROWS_BLK),
        in_specs=[
            pl.BlockSpec((1, ROWS_BLK, EMB), lambda g, r: (g, r, 0)),
            pl.BlockSpec((1, EMB, EMB), lambda g, r: (g, 0, 0)),
            pl.BlockSpec((1, 8, EMB), lambda g, r: (g, 0, 0)),
        ],
        out_specs=pl.BlockSpec((1, ROWS_BLK, EMB), lambda g, r: (g, r, 0)),
        out_shape=jax.ShapeDtypeStruct((NC, N, EMB), jnp.float32),
    )(p, wstack, bpad)


def kernel(embedding_user, embedding_item, Wu0, bu0, Wu1, bu1, Wi0, bi0,
           Wi1, bi1, user_edge_weight, item_edge_weight, user_edge_index,
           item_edge_index):
    src = jnp.concatenate([user_edge_index[0], item_edge_index[0] + N])
    dst = jnp.concatenate([user_edge_index[1], item_edge_index[1]])
    w = jnp.concatenate([user_edge_weight, item_edge_weight])

    x = jnp.concatenate([embedding_user, embedding_item], axis=0)
    weights = [(jnp.stack([Wu0, Wi0]), jnp.stack([bu0, bi0])),
               (jnp.stack([Wu1, Wi1]), jnp.stack([bu1, bi1]))]
    for l in range(2):
        p = _spmm_sc(x, src, dst, w)
        y = _dense_tc(p, weights[l][0], weights[l][1])
        x = y.reshape(NC * N, EMB)
    return (y[0], y[1])


# speedup vs baseline: 2.5368x; 1.6093x over previous
import functools

import jax
import jax.numpy as jnp
from jax import lax
from jax.experimental import pallas as pl
from jax.experimental.pallas import tpu as pltpu
from jax.experimental.pallas import tpu_sc as plsc

N = 10000
EMB = 128
DEG = 32
EG = N * DEG
NC = 2
NS = 16
NW = NC * NS
LANES = 16
K = 80
EPW = EG // NS
NCHUNK = EPW // K
BLK_E = 2000
NBLK = EPW // BLK_E
CPB = BLK_E // K
NZCH = N // K
MAXT = (NZCH + NS - 1) // NS


@functools.partial(
    pl.kernel,
    out_type=jax.ShapeDtypeStruct((NC, N, EMB), jnp.float32),
    mesh=plsc.VectorSubcoreMesh(core_axis_name="c", subcore_axis_name="s",
                                num_cores=NC, num_subcores=NS),
    scratch_types=[
        pltpu.VMEM_SHARED((N, EMB), jnp.float32),
        pltpu.VMEM((BLK_E,), jnp.int32),
        pltpu.VMEM((BLK_E,), jnp.int32),
        pltpu.VMEM((BLK_E,), jnp.float32),
        pltpu.VMEM((K, EMB), jnp.float32),
        pltpu.SemaphoreType.DMA,
    ],
)
def _spmm_sc(x_hbm, src_hbm, dst_hbm, w_hbm, out_hbm, acc_sh,
             src_v, dst_v, w_v, rows_v, semg):
    c = lax.axis_index("c")
    s = lax.axis_index("s")
    ebase = (c * NS + s) * EPW

    def zrow(r, _):
        for j in range(EMB // LANES):
            rows_v[r, pl.ds(j * LANES, LANES)] = jnp.zeros((LANES,),
                                                           jnp.float32)
        return 0
    lax.fori_loop(0, K, zrow, 0)
    for t in range(MAXT):
        idx = s + NS * t
        @pl.when(idx < NZCH)
        def _():
            off = pl.multiple_of(idx * K, 8)
            pltpu.sync_copy(rows_v, acc_sh.at[pl.ds(off, K)])
    plsc.subcore_barrier()

    def block(b, _):
        boff = pl.multiple_of(b * BLK_E, 8)
        pltpu.sync_copy(src_hbm.at[pl.ds(ebase + boff, BLK_E)], src_v)
        pltpu.sync_copy(dst_hbm.at[pl.ds(ebase + boff, BLK_E)], dst_v)
        pltpu.sync_copy(w_hbm.at[pl.ds(ebase + boff, BLK_E)], w_v)

        def chunk(ci, _):
            koff = pl.multiple_of(ci * K, 8)
            pltpu.async_copy(x_hbm.at[src_v.at[pl.ds(koff, K)]], rows_v,
                             semg).wait()

            def group(g, _):
                goff = pl.multiple_of(ci * K + g * LANES, LANES)
                wvec = w_v[pl.ds(goff, LANES)]
                dvec = dst_v[pl.ds(goff, LANES)]
                for el in range(LANES):
                    ws = wvec[el]
                    e = g * LANES + el
                    for j in range(EMB // LANES):
                        rows_v[e, pl.ds(j * LANES, LANES)] = (
                            rows_v[e, pl.ds(j * LANES, LANES)] * ws)
                roff = pl.multiple_of(g * LANES, LANES)
                pltpu.sync_copy(rows_v.at[pl.ds(roff, LANES)],
                                acc_sh.at[dvec], add=True)
                return 0
            lax.fori_loop(0, K // LANES, group, 0)
            return 0
        lax.fori_loop(0, CPB, chunk, 0)
        return 0
    lax.fori_loop(0, NBLK, block, 0)

    plsc.subcore_barrier()
    for t in range(MAXT):
        idx = s + NS * t
        @pl.when(idx < NZCH)
        def _():
            off = pl.multiple_of(idx * K, 8)
            pltpu.sync_copy(acc_sh.at[pl.ds(off, K)],
                            out_hbm.at[c, pl.ds(off, K)])


ROWS_BLK = 2000


def _dense_tc_body(p_ref, w_ref, b_ref, o_ref):
    z = jnp.dot(p_ref[0], w_ref[0], preferred_element_type=jnp.float32)
    z = z + b_ref[0, 0:1, :]
    z = jnp.maximum(z, 0.0)
    nrm = jnp.sqrt(jnp.sum(z * z, axis=1, keepdims=True))
    o_ref[0] = z / jnp.maximum(nrm, 1e-12)


def _dense_tc(p, wstack, bstack):
    bpad = jnp.broadcast_to(bstack[:, None, :], (NC, 8, EMB))
    return pl.pallas_call(
        _dense_tc_body,
        grid=(NC, N // ROWS_BLK),
        in_specs=[
            pl.BlockSpec((1, ROWS_BLK, EMB), lambda g, r: (g, r, 0)),
            pl.BlockSpec((1, EMB, EMB), lambda g, r: (g, 0, 0)),
            pl.BlockSpec((1, 8, EMB), lambda g, r: (g, 0, 0)),
        ],
        out_specs=pl.BlockSpec((1, ROWS_BLK, EMB), lambda g, r: (g, r, 0)),
        out_shape=jax.ShapeDtypeStruct((NC, N, EMB), jnp.float32),
    )(p, wstack, bpad)


def kernel(embedding_user, embedding_item, Wu0, bu0, Wu1, bu1, Wi0, bi0,
           Wi1, bi1, user_edge_weight, item_edge_weight, user_edge_index,
           item_edge_index):
    src = jnp.concatenate([user_edge_index[0], item_edge_index[0] + N])
    dst = jnp.concatenate([user_edge_index[1], item_edge_index[1]])
    w = jnp.concatenate([user_edge_weight, item_edge_weight])

    x = jnp.concatenate([embedding_user, embedding_item], axis=0)
    weights = [(jnp.stack([Wu0, Wi0]), jnp.stack([bu0, bi0])),
               (jnp.stack([Wu1, Wi1]), jnp.stack([bu1, bi1]))]
    for l in range(2):
        p = _spmm_sc(x, src, dst, w)
        y = _dense_tc(p, weights[l][0], weights[l][1])
        x = y.reshape(NC * N, EMB)
    return (y[0], y[1])


# speedup vs baseline: 4.1089x; 1.6197x over previous
import functools

import jax
import jax.numpy as jnp
from jax import lax
from jax.experimental import pallas as pl
from jax.experimental.pallas import tpu as pltpu
from jax.experimental.pallas import tpu_sc as plsc

N = 10000
EMB = 128
DEG = 32
EG = N * DEG
NC = 2
NS = 16
NW = NC * NS
LANES = 16
K = 80
EPW = EG // NS
NCHUNK = EPW // K
BLK_E = 2000
NBLK = EPW // BLK_E
CPB = BLK_E // K
NZCH = N // K
MAXT = (NZCH + NS - 1) // NS


@functools.partial(
    pl.kernel,
    out_type=jax.ShapeDtypeStruct((NC, N, EMB), jnp.float32),
    mesh=plsc.VectorSubcoreMesh(core_axis_name="c", subcore_axis_name="s",
                                num_cores=NC, num_subcores=NS),
    scratch_types=[
        pltpu.VMEM_SHARED((N, EMB), jnp.float32),
        pltpu.VMEM((BLK_E,), jnp.int32),
        pltpu.VMEM((BLK_E,), jnp.int32),
        pltpu.VMEM((BLK_E,), jnp.float32),
        pltpu.VMEM((K, EMB), jnp.float32),
        pltpu.VMEM((K, EMB), jnp.float32),
        pltpu.SemaphoreType.DMA,
        pltpu.SemaphoreType.DMA,
    ],
)
def _spmm_sc(x_hbm, src_hbm, dst_hbm, w_hbm, out_hbm, acc_sh,
             src_v, dst_v, w_v, rows_v, rows_b_v, sema, semb):
    c = lax.axis_index("c")
    s = lax.axis_index("s")
    ebase = (c * NS + s) * EPW

    def zrow(r, _):
        for j in range(EMB // LANES):
            rows_v[r, pl.ds(j * LANES, LANES)] = jnp.zeros((LANES,),
                                                           jnp.float32)
        return 0
    lax.fori_loop(0, K, zrow, 0)
    for t in range(MAXT):
        idx = s + NS * t
        @pl.when(idx < NZCH)
        def _():
            off = pl.multiple_of(idx * K, 8)
            pltpu.sync_copy(rows_v, acc_sh.at[pl.ds(off, K)])
    plsc.subcore_barrier()

    def block(b, _):
        boff = pl.multiple_of(b * BLK_E, 8)
        pltpu.sync_copy(src_hbm.at[pl.ds(ebase + boff, BLK_E)], src_v)
        pltpu.sync_copy(dst_hbm.at[pl.ds(ebase + boff, BLK_E)], dst_v)
        pltpu.sync_copy(w_hbm.at[pl.ds(ebase + boff, BLK_E)], w_v)

        def issue(ci, buf, sem):
            koff = pl.multiple_of(ci * K, 8)
            pltpu.async_copy(x_hbm.at[src_v.at[pl.ds(koff, K)]], buf, sem)

        def drain(buf, sem):
            pltpu.make_async_copy(x_hbm.at[pl.ds(0, K)], buf, sem).wait()

        def proc(ci, buf):
            def group(g, _):
                goff = pl.multiple_of(ci * K + g * LANES, LANES)
                wvec = w_v[pl.ds(goff, LANES)]
                dvec = dst_v[pl.ds(goff, LANES)]
                for el in range(LANES):
                    ws = wvec[el]
                    e = g * LANES + el
                    for j in range(EMB // LANES):
                        buf[e, pl.ds(j * LANES, LANES)] = (
                            buf[e, pl.ds(j * LANES, LANES)] * ws)
                roff = pl.multiple_of(g * LANES, LANES)
                pltpu.sync_copy(buf.at[pl.ds(roff, LANES)],
                                acc_sh.at[dvec], add=True)
                return 0
            lax.fori_loop(0, K // LANES, group, 0)

        issue(0, rows_v, sema)
        def pair(h, _):
            ca = 2 * h
            drain(rows_v, sema)
            issue(ca + 1, rows_b_v, semb)
            proc(ca, rows_v)
            drain(rows_b_v, semb)
            issue(ca + 2, rows_v, sema)
            proc(ca + 1, rows_b_v)
            return 0
        lax.fori_loop(0, (CPB - 1) // 2, pair, 0)
        drain(rows_v, sema)
        proc(CPB - 1, rows_v)
        return 0
    lax.fori_loop(0, NBLK, block, 0)

    plsc.subcore_barrier()
    for t in range(MAXT):
        idx = s + NS * t
        @pl.when(idx < NZCH)
        def _():
            off = pl.multiple_of(idx * K, 8)
            pltpu.sync_copy(acc_sh.at[pl.ds(off, K)],
                            out_hbm.at[c, pl.ds(off, K)])


ROWS_BLK = 2000


def _dense_tc_body(p_ref, w_ref, b_ref, o_ref):
    z = jnp.dot(p_ref[0], w_ref[0], preferred_element_type=jnp.float32)
    z = z + b_ref[0, 0:1, :]
    z = jnp.maximum(z, 0.0)
    nrm = jnp.sqrt(jnp.sum(z * z, axis=1, keepdims=True))
    o_ref[0] = z / jnp.maximum(nrm, 1e-12)


def _dense_tc(p, wstack, bstack):
    bpad = jnp.broadcast_to(bstack[:, None, :], (NC, 8, EMB))
    return pl.pallas_call(
        _dense_tc_body,
        grid=(NC, N // ROWS_BLK),
        in_specs=[
            pl.BlockSpec((1, ROWS_BLK, EMB), lambda g, r: (g, r, 0)),
            pl.BlockSpec((1, EMB, EMB), lambda g, r: (g, 0, 0)),
            pl.BlockSpec((1, 8, EMB), lambda g, r: (g, 0, 0)),
        ],
        out_specs=pl.BlockSpec((1, ROWS_BLK, EMB), lambda g, r: (g, r, 0)),
        out_shape=jax.ShapeDtypeStruct((NC, N, EMB), jnp.float32),
    )(p, wstack, bpad)


def kernel(embedding_user, embedding_item, Wu0, bu0, Wu1, bu1, Wi0, bi0,
           Wi1, bi1, user_edge_weight, item_edge_weight, user_edge_index,
           item_edge_index):
    src = jnp.concatenate([user_edge_index[0], item_edge_index[0] + N])
    dst = jnp.concatenate([user_edge_index[1], item_edge_index[1]])
    w = jnp.concatenate([user_edge_weight, item_edge_weight])

    x = jnp.concatenate([embedding_user, embedding_item], axis=0)
    weights = [(jnp.stack([Wu0, Wi0]), jnp.stack([bu0, bi0])),
               (jnp.stack([Wu1, Wi1]), jnp.stack([bu1, bi1]))]
    for l in range(2):
        p = _spmm_sc(x, src, dst, w)
        y = _dense_tc(p, weights[l][0], weights[l][1])
        x = y.reshape(NC * N, EMB)
    return (y[0], y[1])
